# baseline (speedup 1.0000x reference)
import functools

import jax
import jax.numpy as jnp
from jax import lax
from jax.experimental import pallas as pl

_K = 32


def _count_ge(bits, mid):
    return jnp.sum((bits >= mid).astype(jnp.int32), axis=1, keepdims=True)


def _enc_body(x_ref, wt_ref, b_ref, z_ref):
    z = (
        lax.dot_general(
            x_ref[...],
            wt_ref[...],
            (((1,), (1,)), ((), ())),
            preferred_element_type=jnp.float32,
        )
        + b_ref[...]
    )
    bits = lax.bitcast_convert_type(z, jnp.int32) & 0x7FFFFFFF
    rb, dd = z.shape

    m = jnp.max(bits.reshape(rb, dd // 128, 128), axis=2)
    rowmax = jnp.max(m, axis=1, keepdims=True)
    hi0 = rowmax + 1

    def mbody(_, carry):
        lo, hi = carry
        mid = lo + ((hi - lo) >> 1)
        cnt = jnp.sum((m >= mid).astype(jnp.int32), axis=1, keepdims=True)
        ge = cnt >= _K + 1
        return jnp.where(ge, mid, lo), jnp.where(ge, hi, mid)

    lo_m, _ = lax.fori_loop(0, 14, mbody, (jnp.zeros((rb, 1), jnp.int32), hi0))

    def cond(carry):
        _, _, _, active = carry
        return jnp.max(active) > 0

    def body(carry):
        lo, hi, thr, active = carry
        act = active > 0
        mid = lo + ((hi - lo) >> 1)
        cnt = _count_ge(bits, mid)
        hit = act & (cnt == _K)
        thr = jnp.where(hit, mid, thr)
        ge = cnt >= _K
        lo = jnp.where(act & ge, mid, lo)
        hi = jnp.where(act & ~ge, mid, hi)
        active = (act & ~hit & ((hi - lo) > 1)).astype(jnp.int32)
        return lo, hi, thr, active

    lo0 = lo_m
    thr0 = jnp.full((rb, 1), -1, jnp.int32)
    active0 = jnp.ones((rb, 1), jnp.int32)
    lo, _, thr, _ = lax.while_loop(cond, body, (lo0, hi0, thr0, active0))
    thr = jnp.where(thr >= 0, thr, lo)
    z_ref[...] = jnp.where(bits >= thr, z, 0.0)


def _dec_body(z_ref, wt_ref, b_ref, o_ref):
    o_ref[...] = (
        lax.dot_general(
            z_ref[...].astype(jnp.bfloat16),
            wt_ref[...],
            (((1,), (1,)), ((), ())),
            preferred_element_type=jnp.float32,
        )
        + b_ref[...]
    )


@jax.jit
def kernel(x, W_enc, b_enc, W_dec, b_dec):
    n, d_model = x.shape
    d_dict = W_enc.shape[0]
    rb = 64
    grid = (n // rb,)

    z = pl.pallas_call(
        _enc_body,
        grid=grid,
        in_specs=[
            pl.BlockSpec((rb, d_model), lambda i: (i, 0)),
            pl.BlockSpec((d_dict, d_model), lambda i: (0, 0)),
            pl.BlockSpec((1, d_dict), lambda i: (0, 0)),
        ],
        out_specs=pl.BlockSpec((rb, d_dict), lambda i: (i, 0)),
        out_shape=jax.ShapeDtypeStruct((n, d_dict), jnp.float32),
    )(x, W_enc, b_enc.reshape(1, d_dict))

    rb2 = 256
    out = pl.pallas_call(
        _dec_body,
        grid=(n // rb2,),
        in_specs=[
            pl.BlockSpec((rb2, d_dict), lambda i: (i, 0)),
            pl.BlockSpec((d_model, d_dict), lambda i: (0, 0)),
            pl.BlockSpec((1, d_model), lambda i: (0, 0)),
        ],
        out_specs=pl.BlockSpec((rb2, d_model), lambda i: (i, 0)),
        out_shape=jax.ShapeDtypeStruct((n, d_model), jnp.float32),
    )(z, W_dec.astype(jnp.bfloat16), b_dec.reshape(1, d_model))

    return (out, z)
